# baseline (speedup 1.0000x reference)
import functools

import numpy as np

import jax
import jax.numpy as jnp
from jax import lax
from jax.experimental import pallas as pl
from jax.experimental.pallas import tpu as pltpu
from jax.experimental.pallas import tpu_sc as plsc

N = 10000
E = 320000
IN_CH = 128
HID = 64
TIME_DIM = 16
OUT_CH = 128

NW = 32
CH = 128
CPW = 80
EP = NW * CPW * CH
NR = 10240
RPT = NR // 16


_BR = 512


def _tp_body(t_ref, w_ref, b_ref, w0_ref, b0_ref, w1_ref, b1_ref,
             tp0_ref, tp1_ref):
    te = jnp.cos(t_ref[...] * w_ref[...] + b_ref[...])
    for q in range(4):
        r = pl.ds(q * _BR, _BR)
        tp0_ref[r, :] = (jnp.dot(te, w0_ref[q],
                                 preferred_element_type=jnp.float32)
                         + b0_ref[...]).astype(jnp.bfloat16)
        tp1_ref[r, :] = (jnp.dot(te, w1_ref[q],
                                 preferred_element_type=jnp.float32)
                         + b1_ref[...]).astype(jnp.bfloat16)


def _tp_call(t_rep, w_tile, b_tile, w0_q, b0_2, w1_q, b1_2):
    rows = EP // 2
    g = EP // (8 * _BR)
    return pl.pallas_call(
        _tp_body,
        grid=(g,),
        in_specs=[
            pl.BlockSpec((_BR, 128), lambda i: (i, 0)),
            pl.BlockSpec((1, 128), lambda i: (0, 0)),
            pl.BlockSpec((1, 128), lambda i: (0, 0)),
            pl.BlockSpec((4, 128, 128), lambda i: (0, 0, 0)),
            pl.BlockSpec((1, 128), lambda i: (0, 0)),
            pl.BlockSpec((4, 128, 128), lambda i: (0, 0, 0)),
            pl.BlockSpec((1, 128), lambda i: (0, 0)),
        ],
        out_specs=[
            pl.BlockSpec((4 * _BR, 128), lambda i: (i, 0)),
            pl.BlockSpec((4 * _BR, 128), lambda i: (i, 0)),
        ],
        out_shape=[
            jax.ShapeDtypeStruct((rows, 128), jnp.bfloat16),
            jax.ShapeDtypeStruct((rows, 128), jnp.bfloat16),
        ],
    )(t_rep, w_tile, b_tile, w0_q, b0_2, w1_q, b1_2)


def _mm_body(x_ref, w_ref, o_ref):
    o_ref[...] = jnp.dot(x_ref[...], w_ref[...],
                         preferred_element_type=jnp.float32
                         ).astype(jnp.bfloat16)


def _mm_call(x, w):
    m, k = x.shape
    _, n = w.shape
    return pl.pallas_call(
        _mm_body,
        out_shape=jax.ShapeDtypeStruct((m, n), jnp.bfloat16),
    )(x, w)


def _mid_body(a_ref, w1x_ref, y1_ref, deg_ref):
    p = a_ref[0] + a_ref[1]
    deg = jnp.maximum(p[:, HID:HID + 1], 1.0)
    h = p[:, :HID] / deg
    y1_ref[...] = jnp.dot(h, w1x_ref[...],
                          preferred_element_type=jnp.float32
                          ).astype(jnp.bfloat16)
    deg_ref[...] = deg


def _mid_call(agg0, w1x):
    return pl.pallas_call(
        _mid_body,
        out_shape=[
            jax.ShapeDtypeStruct((NR, HID), jnp.bfloat16),
            jax.ShapeDtypeStruct((NR, 1), jnp.float32),
        ],
    )(agg0, w1x)


def _fin_body(a_ref, deg_ref, wc_ref, bc_ref, o_ref):
    p = a_ref[0] + a_ref[1]
    h = p / deg_ref[...]
    o_ref[...] = jnp.dot(h, wc_ref[...],
                         preferred_element_type=jnp.float32) + bc_ref[...]


def _fin_call(agg1, deg, wc, bc):
    return pl.pallas_call(
        _fin_body,
        out_shape=jax.ShapeDtypeStruct((NR, OUT_CH), jnp.float32),
    )(agg1, deg, wc, bc)


def _make_sc_layer(dw):
    mesh = plsc.VectorSubcoreMesh(core_axis_name="c", subcore_axis_name="s")

    @functools.partial(
        pl.kernel,
        out_type=jax.ShapeDtypeStruct((2, NR, dw), jnp.float32),
        mesh=mesh,
        compiler_params=pltpu.CompilerParams(use_tc_tiling_on_sc=False,
                                             needs_layout_passes=False),
        scratch_types=[
            pltpu.VMEM((CPW, CH), jnp.int32),
            pltpu.VMEM((CPW, CH), jnp.int32),
            pltpu.VMEM((2, CH // 2, 128), jnp.bfloat16),
            pltpu.VMEM((2, CH, HID), jnp.bfloat16),
            pltpu.VMEM((2, CH, dw), jnp.float32),
            pltpu.VMEM_SHARED((NR, dw), jnp.float32),
            pltpu.VMEM_SHARED((NR, HID), jnp.bfloat16),
            pltpu.SemaphoreType.DMA,
            pltpu.SemaphoreType.DMA,
            pltpu.SemaphoreType.DMA,
            pltpu.SemaphoreType.DMA,
            pltpu.SemaphoreType.DMA,
            pltpu.SemaphoreType.DMA,
        ],
    )
    def sc_layer(y_hbm, tp_hbm, src_hbm, dst_hbm, out_hbm,
                 srcv, dstv, tpv, gv, msgv, aggsh, ysh,
                 sem_t0, sem_t1, sem_g0, sem_g1, sem_s0, sem_s1):
        ci = lax.axis_index("c")
        si = lax.axis_index("s")
        w = ci * 16 + si
        nk = dw // 16

        @pl.loop(0, CH)
        def _(i):
            for k in range(nk):
                msgv[0, i, pl.ds(16 * k, 16)] = jnp.zeros((16,), jnp.float32)

        @pl.loop(0, RPT // CH)
        def _(i):
            base = (si * (RPT // CH) + i) * CH
            pltpu.sync_copy(msgv.at[0], aggsh.at[pl.ds(base, CH)])

        pltpu.sync_copy(src_hbm.at[w], srcv)
        pltpu.sync_copy(dst_hbm.at[w], dstv)

        @pl.loop(0, RPT // CH)
        def _(i):
            rows = pl.ds((si * (RPT // CH) + i) * CH, CH)
            pltpu.sync_copy(y_hbm.at[rows], gv.at[0])
            pltpu.sync_copy(gv.at[0], ysh.at[rows])

        if dw > HID:
            one0 = jnp.where(lax.iota(jnp.int32, 16) == 0,
                             1.0, 0.0).astype(jnp.float32)
            zer = jnp.zeros((16,), jnp.float32)

            @pl.loop(0, CH)
            def _(i):
                for b in range(2):
                    msgv[b, i, pl.ds(HID, 16)] = one0
                    for k in range(HID // 16 + 1, nk):
                        msgv[b, i, pl.ds(16 * k, 16)] = zer

        plsc.subcore_barrier()

        sems_t = (sem_t0, sem_t1)
        sems_g = (sem_g0, sem_g1)
        sems_s = (sem_s0, sem_s1)

        def start(j, b):
            pltpu.async_copy(tp_hbm.at[pl.ds((w * CPW + j) * (CH // 2),
                                             CH // 2)],
                             tpv.at[b], sems_t[b])
            pltpu.async_copy(ysh.at[srcv.at[j]], gv.at[b], sems_g[b])

        def drain_scatter(b):
            pltpu.make_async_copy(out_hbm.at[0, pl.ds(0, CH)],
                                  msgv.at[b], sems_s[b]).wait()

        start(0, 0)

        @pl.loop(0, CPW, step=2)
        def _(g):
            for b in range(2):
                j = g + b
                nj = j + 1

                @pl.when(nj < CPW)
                def _():
                    start(nj, 1 - b)

                pltpu.make_async_copy(tp_hbm.at[pl.ds(0, CH // 2)],
                                      tpv.at[b], sems_t[b]).wait()
                pltpu.make_async_copy(y_hbm.at[pl.ds(0, CH)],
                                      gv.at[b], sems_g[b]).wait()

                @pl.when(j >= 2)
                def _():
                    drain_scatter(b)

                @plsc.parallel_loop(0, CH // 2, unroll=2)
                def _(i2):
                    for eh in range(2):
                        i = 2 * i2 + eh
                        for k in range(HID // 32):
                            t_lo, t_hi = plsc.unpack(
                                tpv[b, i2, pl.ds(64 * eh + 32 * k, 32)],
                                format=plsc.PackFormat.INTERLEAVED)
                            g_lo, g_hi = plsc.unpack(
                                gv[b, i, pl.ds(32 * k, 32)],
                                format=plsc.PackFormat.INTERLEAVED)
                            msgv[b, i, pl.ds(32 * k, 16)] = jnp.maximum(
                                g_lo + t_lo, 0.0)
                            msgv[b, i, pl.ds(32 * k + 16, 16)] = jnp.maximum(
                                g_hi + t_hi, 0.0)

                pltpu.async_copy(msgv.at[b], aggsh.at[dstv.at[j]],
                                 sems_s[b], add=True)

        drain_scatter(0)
        drain_scatter(1)
        plsc.subcore_barrier()

        @pl.loop(0, RPT // CH)
        def _(i):
            base = (si * (RPT // CH) + i) * CH
            pltpu.sync_copy(aggsh.at[pl.ds(base, CH)], msgv.at[0])
            pltpu.sync_copy(msgv.at[0], out_hbm.at[ci, pl.ds(base, CH)])

    return sc_layer


_sc_layer0 = _make_sc_layer(80)
_sc_layer1 = _make_sc_layer(64)


_BF16_PERM32 = np.empty(32, np.int32)
_BF16_PERM32[0::2] = np.arange(16)
_BF16_PERM32[1::2] = np.arange(16) + 16
_BF16_PERM64 = np.concatenate([_BF16_PERM32, _BF16_PERM32 + 32])


def _edge_perm(a):
    return a.reshape(EP // 4096, 512, 4, 2).transpose(0, 2, 1, 3).reshape(-1)


def kernel(x, edge_index, timestamps, time_w, time_b, W0, b0, W1, b1, Wc, bc):
    pad = EP - E
    src = edge_index[0].astype(jnp.int32)
    dst = edge_index[1].astype(jnp.int32)
    src_p = _edge_perm(jnp.concatenate(
        [src, jnp.zeros((pad,), jnp.int32)])).reshape(NW, CPW, CH)
    dst_p = _edge_perm(jnp.concatenate(
        [dst, N + (jnp.arange(pad, dtype=jnp.int32) % (NR - N))]
    )).reshape(NW, CPW, CH)
    ts_p = jnp.concatenate(
        [timestamps.astype(jnp.float32), jnp.zeros((pad,), jnp.float32)])
    t_rep = jnp.broadcast_to(ts_p[:, None], (EP, TIME_DIM)).reshape(EP // 8, 128)

    w0x, w0t = W0[:IN_CH], W0[IN_CH:]
    w1x, w1t = W1[:HID], W1[HID:]
    p64 = _BF16_PERM64
    w0xp = w0x[:, p64]
    w1xp = w1x[:, p64]
    w0tp = w0t[:, p64]
    w1tp = w1t[:, p64]
    b0p = b0[p64]
    b1p = b1[p64]

    def qweights(wt):
        wq = jnp.zeros((4, 128, 128), jnp.float32)
        for q in range(4):
            wq = wq.at[q, 32 * q:32 * q + 16, :HID].set(wt)
            wq = wq.at[q, 32 * q + 16:32 * q + 32, HID:].set(wt)
        return wq

    w0_q = qweights(w0tp)
    w1_q = qweights(w1tp)
    w_tile = jnp.tile(time_w.astype(jnp.float32), 8).reshape(1, 128)
    b_tile = jnp.tile(time_b.astype(jnp.float32), 8).reshape(1, 128)
    b0_2 = jnp.tile(b0p.astype(jnp.float32), 2).reshape(1, 128)
    b1_2 = jnp.tile(b1p.astype(jnp.float32), 2).reshape(1, 128)

    tp0, tp1 = _tp_call(t_rep, w_tile, b_tile, w0_q, b0_2, w1_q, b1_2)
    x_pad = jnp.concatenate(
        [x, jnp.zeros((NR - N, IN_CH), jnp.float32)])
    y0 = _mm_call(x_pad, w0xp)

    agg0 = _sc_layer0(y0, tp0, src_p, dst_p)
    y1, deg = _mid_call(agg0, w1xp)
    agg1 = _sc_layer1(y1, tp1, src_p, dst_p)
    out = _fin_call(agg1, deg, Wc, bc.reshape(1, OUT_CH))
    return out[:N]


# speedup vs baseline: 1.5679x; 1.5679x over previous
import functools

import numpy as np

import jax
import jax.numpy as jnp
from jax import lax
from jax.experimental import pallas as pl
from jax.experimental.pallas import tpu as pltpu
from jax.experimental.pallas import tpu_sc as plsc

N = 10000
E = 320000
IN_CH = 128
HID = 64
TIME_DIM = 16
OUT_CH = 128

NW = 32
CH = 128
CPW = 80
EP = NW * CPW * CH
NR = 10240
RPT = NR // 16


_BR = 512


def _tp_body(t_ref, w_ref, b_ref, w0_ref, b0_ref, tp0_ref):
    te = jnp.cos(t_ref[...] * w_ref[...] + b_ref[...])
    tp0_ref[...] = (jnp.dot(te, w0_ref[...], preferred_element_type=jnp.float32)
                    + b0_ref[...]).astype(jnp.bfloat16)


def _tp_call(t_rep, w_tile, b_tile, w_blk, b_blk):
    rows = EP // 8
    g = rows // _BR
    return pl.pallas_call(
        _tp_body,
        grid=(g,),
        in_specs=[
            pl.BlockSpec((_BR, 128), lambda i: (i, 0)),
            pl.BlockSpec((1, 128), lambda i: (0, 0)),
            pl.BlockSpec((1, 128), lambda i: (0, 0)),
            pl.BlockSpec((128, 512), lambda i: (0, 0)),
            pl.BlockSpec((1, 512), lambda i: (0, 0)),
        ],
        out_specs=[
            pl.BlockSpec((_BR, 512), lambda i: (i, 0)),
        ],
        out_shape=[
            jax.ShapeDtypeStruct((rows, 512), jnp.bfloat16),
        ],
    )(t_rep, w_tile, b_tile, w_blk, b_blk)


def _mm_body(x_ref, w_ref, o_ref):
    o_ref[...] = jnp.dot(x_ref[...], w_ref[...],
                         preferred_element_type=jnp.float32
                         ).astype(jnp.bfloat16)


def _mm_call(x, w):
    m, k = x.shape
    _, n = w.shape
    return pl.pallas_call(
        _mm_body,
        out_shape=jax.ShapeDtypeStruct((m, n), jnp.bfloat16),
    )(x, w)


def _mid_body(a_ref, w1x_ref, y1_ref, deg_ref):
    p = a_ref[0] + a_ref[1]
    deg = jnp.maximum(p[:, HID:HID + 1], 1.0)
    h = p[:, :HID] / deg
    y1_ref[...] = jnp.dot(h, w1x_ref[...],
                          preferred_element_type=jnp.float32
                          ).astype(jnp.bfloat16)
    deg_ref[...] = deg


def _mid_call(agg0, w1x):
    return pl.pallas_call(
        _mid_body,
        out_shape=[
            jax.ShapeDtypeStruct((NR, HID), jnp.bfloat16),
            jax.ShapeDtypeStruct((NR, 1), jnp.float32),
        ],
    )(agg0, w1x)


def _fin_body(a_ref, deg_ref, wc_ref, bc_ref, o_ref):
    p = a_ref[0] + a_ref[1]
    h = p / deg_ref[...]
    o_ref[...] = jnp.dot(h, wc_ref[...],
                         preferred_element_type=jnp.float32) + bc_ref[...]


def _fin_call(agg1, deg, wc, bc):
    return pl.pallas_call(
        _fin_body,
        out_shape=jax.ShapeDtypeStruct((NR, OUT_CH), jnp.float32),
    )(agg1, deg, wc, bc)


def _make_sc_layer(dw):
    mesh = plsc.VectorSubcoreMesh(core_axis_name="c", subcore_axis_name="s")

    @functools.partial(
        pl.kernel,
        out_type=jax.ShapeDtypeStruct((2, NR, dw), jnp.float32),
        mesh=mesh,
        compiler_params=pltpu.CompilerParams(use_tc_tiling_on_sc=False,
                                             needs_layout_passes=False),
        scratch_types=[
            pltpu.VMEM((CPW, CH), jnp.int32),
            pltpu.VMEM((CPW, CH), jnp.int32),
            pltpu.VMEM((2, CH, HID), jnp.bfloat16),
            pltpu.VMEM((2, CH, HID), jnp.bfloat16),
            pltpu.VMEM((2, CH, dw), jnp.float32),
            pltpu.VMEM_SHARED((NR, dw), jnp.float32),
            pltpu.VMEM_SHARED((NR, HID), jnp.bfloat16),
            pltpu.SemaphoreType.DMA,
            pltpu.SemaphoreType.DMA,
            pltpu.SemaphoreType.DMA,
            pltpu.SemaphoreType.DMA,
            pltpu.SemaphoreType.DMA,
            pltpu.SemaphoreType.DMA,
        ],
    )
    def sc_layer(y_hbm, tp_hbm, src_hbm, dst_hbm, out_hbm,
                 srcv, dstv, tpv, gv, msgv, aggsh, ysh,
                 sem_t0, sem_t1, sem_g0, sem_g1, sem_s0, sem_s1):
        ci = lax.axis_index("c")
        si = lax.axis_index("s")
        w = ci * 16 + si
        nk = dw // 16

        @pl.loop(0, CH)
        def _(i):
            for k in range(nk):
                msgv[0, i, pl.ds(16 * k, 16)] = jnp.zeros((16,), jnp.float32)

        @pl.loop(0, RPT // CH)
        def _(i):
            base = (si * (RPT // CH) + i) * CH
            pltpu.sync_copy(msgv.at[0], aggsh.at[pl.ds(base, CH)])

        pltpu.sync_copy(src_hbm.at[w], srcv)
        pltpu.sync_copy(dst_hbm.at[w], dstv)

        @pl.loop(0, RPT // CH)
        def _(i):
            rows = pl.ds((si * (RPT // CH) + i) * CH, CH)
            pltpu.sync_copy(y_hbm.at[rows], gv.at[0])
            pltpu.sync_copy(gv.at[0], ysh.at[rows])

        if dw > HID:
            one0 = jnp.where(lax.iota(jnp.int32, 16) == 0,
                             1.0, 0.0).astype(jnp.float32)
            zer = jnp.zeros((16,), jnp.float32)

            @pl.loop(0, CH)
            def _(i):
                for b in range(2):
                    msgv[b, i, pl.ds(HID, 16)] = one0
                    for k in range(HID // 16 + 1, nk):
                        msgv[b, i, pl.ds(16 * k, 16)] = zer

        plsc.subcore_barrier()

        sems_t = (sem_t0, sem_t1)
        sems_g = (sem_g0, sem_g1)
        sems_s = (sem_s0, sem_s1)

        def start(j, b):
            pltpu.async_copy(tp_hbm.at[pl.ds((w * CPW + j) * CH, CH)],
                             tpv.at[b], sems_t[b])
            pltpu.async_copy(ysh.at[srcv.at[j]], gv.at[b], sems_g[b])

        def drain_scatter(b):
            pltpu.make_async_copy(out_hbm.at[0, pl.ds(0, CH)],
                                  msgv.at[b], sems_s[b]).wait()

        start(0, 0)

        @pl.loop(0, CPW, step=2)
        def _(g):
            for b in range(2):
                j = g + b
                nj = j + 1

                @pl.when(nj < CPW)
                def _():
                    start(nj, 1 - b)

                pltpu.make_async_copy(tp_hbm.at[pl.ds(0, CH)],
                                      tpv.at[b], sems_t[b]).wait()
                pltpu.make_async_copy(y_hbm.at[pl.ds(0, CH)],
                                      gv.at[b], sems_g[b]).wait()

                @pl.when(j >= 2)
                def _():
                    drain_scatter(b)

                @plsc.parallel_loop(0, CH, unroll=4)
                def _(i):
                    for k in range(HID // 32):
                        s32 = pl.ds(32 * k, 32)
                        t_lo, t_hi = plsc.unpack(
                            tpv[b, i, s32], format=plsc.PackFormat.INTERLEAVED)
                        g_lo, g_hi = plsc.unpack(
                            gv[b, i, s32], format=plsc.PackFormat.INTERLEAVED)
                        msgv[b, i, pl.ds(32 * k, 16)] = jnp.maximum(
                            g_lo + t_lo, 0.0)
                        msgv[b, i, pl.ds(32 * k + 16, 16)] = jnp.maximum(
                            g_hi + t_hi, 0.0)

                pltpu.async_copy(msgv.at[b], aggsh.at[dstv.at[j]],
                                 sems_s[b], add=True)

        drain_scatter(0)
        drain_scatter(1)
        plsc.subcore_barrier()

        @pl.loop(0, RPT // CH)
        def _(i):
            base = (si * (RPT // CH) + i) * CH
            pltpu.sync_copy(aggsh.at[pl.ds(base, CH)], msgv.at[0])
            pltpu.sync_copy(msgv.at[0], out_hbm.at[ci, pl.ds(base, CH)])

    return sc_layer


_sc_layer0 = _make_sc_layer(80)
_sc_layer1 = _make_sc_layer(64)


_BF16_PERM32 = np.empty(32, np.int32)
_BF16_PERM32[0::2] = np.arange(16)
_BF16_PERM32[1::2] = np.arange(16) + 16
_BF16_PERM64 = np.concatenate([_BF16_PERM32, _BF16_PERM32 + 32])


def kernel(x, edge_index, timestamps, time_w, time_b, W0, b0, W1, b1, Wc, bc):
    pad = EP - E
    src = edge_index[0].astype(jnp.int32)
    dst = edge_index[1].astype(jnp.int32)
    src_p = jnp.concatenate(
        [src, jnp.zeros((pad,), jnp.int32)]).reshape(NW, CPW, CH)
    dst_p = jnp.concatenate(
        [dst, N + (jnp.arange(pad, dtype=jnp.int32) % (NR - N))]
    ).reshape(NW, CPW, CH)
    ts_p = jnp.concatenate(
        [timestamps.astype(jnp.float32), jnp.zeros((pad,), jnp.float32)])
    t_rep = jnp.broadcast_to(ts_p[:, None], (EP, TIME_DIM)).reshape(EP // 8, 128)

    w0x, w0t = W0[:IN_CH], W0[IN_CH:]
    w1x, w1t = W1[:HID], W1[HID:]
    p64 = _BF16_PERM64
    w0xp = w0x[:, p64]
    w1xp = w1x[:, p64]
    w0tp = w0t[:, p64]
    w1tp = w1t[:, p64]
    b0p = b0[p64]
    b1p = b1[p64]
    eye8 = jnp.eye(8, dtype=jnp.float32)
    w0_blk = jnp.kron(eye8, w0tp)
    w1_blk = jnp.kron(eye8, w1tp)
    w_tile = jnp.tile(time_w.astype(jnp.float32), 8).reshape(1, 128)
    b_tile = jnp.tile(time_b.astype(jnp.float32), 8).reshape(1, 128)
    b0_blk = jnp.tile(b0p.astype(jnp.float32), 8).reshape(1, 512)
    b1_blk = jnp.tile(b1p.astype(jnp.float32), 8).reshape(1, 512)

    tp0m, = _tp_call(t_rep, w_tile, b_tile, w0_blk, b0_blk)
    tp1m, = _tp_call(t_rep, w_tile, b_tile, w1_blk, b1_blk)
    tp0 = tp0m.reshape(EP, HID)
    tp1 = tp1m.reshape(EP, HID)
    x_pad = jnp.concatenate(
        [x, jnp.zeros((NR - N, IN_CH), jnp.float32)])
    y0 = _mm_call(x_pad, w0xp)

    agg0 = _sc_layer0(y0, tp0, src_p, dst_p)
    y1, deg = _mid_call(agg0, w1xp)
    agg1 = _sc_layer1(y1, tp1, src_p, dst_p)
    out = _fin_call(agg1, deg, Wc, bc.reshape(1, OUT_CH))
    return out[:N]


# speedup vs baseline: 1.7640x; 1.1251x over previous
import functools

import numpy as np

import jax
import jax.numpy as jnp
from jax import lax
from jax.experimental import pallas as pl
from jax.experimental.pallas import tpu as pltpu
from jax.experimental.pallas import tpu_sc as plsc

N = 10000
E = 320000
IN_CH = 128
HID = 64
TIME_DIM = 16
OUT_CH = 128

NW = 32
CH = 128
CPW = 80
EP = NW * CPW * CH
NR = 10240
RPT = NR // 16


_BR = 512


def _tp_body(t_ref, w_ref, b_ref, w0_ref, b0_ref, w1_ref, b1_ref,
             tp0_ref, tp1_ref):
    te = jnp.cos(t_ref[...] * w_ref[...] + b_ref[...])
    tp0_ref[...] = (jnp.dot(te, w0_ref[...], preferred_element_type=jnp.float32)
                    + b0_ref[...]).astype(jnp.bfloat16)
    tp1_ref[...] = (jnp.dot(te, w1_ref[...], preferred_element_type=jnp.float32)
                    + b1_ref[...]).astype(jnp.bfloat16)


def _tp_call(t_rep, w_tile, b_tile, w0_blk, b0_blk, w1_blk, b1_blk):
    rows = EP // 8
    g = rows // _BR
    return pl.pallas_call(
        _tp_body,
        grid=(g,),
        in_specs=[
            pl.BlockSpec((_BR, 128), lambda i: (i, 0)),
            pl.BlockSpec((1, 128), lambda i: (0, 0)),
            pl.BlockSpec((1, 128), lambda i: (0, 0)),
            pl.BlockSpec((128, 512), lambda i: (0, 0)),
            pl.BlockSpec((1, 512), lambda i: (0, 0)),
            pl.BlockSpec((128, 512), lambda i: (0, 0)),
            pl.BlockSpec((1, 512), lambda i: (0, 0)),
        ],
        out_specs=[
            pl.BlockSpec((_BR, 512), lambda i: (i, 0)),
            pl.BlockSpec((_BR, 512), lambda i: (i, 0)),
        ],
        out_shape=[
            jax.ShapeDtypeStruct((rows, 512), jnp.bfloat16),
            jax.ShapeDtypeStruct((rows, 512), jnp.bfloat16),
        ],
    )(t_rep, w_tile, b_tile, w0_blk, b0_blk, w1_blk, b1_blk)


def _mm_body(x_ref, w_ref, o_ref):
    o_ref[...] = jnp.dot(x_ref[...], w_ref[...],
                         preferred_element_type=jnp.float32
                         ).astype(jnp.bfloat16)


def _mm_call(x, w):
    m, k = x.shape
    _, n = w.shape
    return pl.pallas_call(
        _mm_body,
        out_shape=jax.ShapeDtypeStruct((m, n), jnp.bfloat16),
    )(x, w)


def _mid_body(a_ref, w1x_ref, y1_ref, deg_ref):
    p = a_ref[0] + a_ref[1]
    deg = jnp.maximum(p[:, HID:HID + 1], 1.0)
    h = p[:, :HID] / deg
    y1_ref[...] = jnp.dot(h, w1x_ref[...],
                          preferred_element_type=jnp.float32
                          ).astype(jnp.bfloat16)
    deg_ref[...] = deg


def _mid_call(agg0, w1x):
    return pl.pallas_call(
        _mid_body,
        out_shape=[
            jax.ShapeDtypeStruct((NR, HID), jnp.bfloat16),
            jax.ShapeDtypeStruct((NR, 1), jnp.float32),
        ],
    )(agg0, w1x)


def _fin_body(a_ref, deg_ref, wc_ref, bc_ref, o_ref):
    p = a_ref[0] + a_ref[1]
    h = p / deg_ref[...]
    o_ref[...] = jnp.dot(h, wc_ref[...],
                         preferred_element_type=jnp.float32) + bc_ref[...]


def _fin_call(agg1, deg, wc, bc):
    return pl.pallas_call(
        _fin_body,
        out_shape=jax.ShapeDtypeStruct((NR, OUT_CH), jnp.float32),
    )(agg1, deg, wc, bc)


def _make_sc_layer(dw):
    mesh = plsc.VectorSubcoreMesh(core_axis_name="c", subcore_axis_name="s")

    @functools.partial(
        pl.kernel,
        out_type=jax.ShapeDtypeStruct((2, NR, dw), jnp.float32),
        mesh=mesh,
        compiler_params=pltpu.CompilerParams(use_tc_tiling_on_sc=False,
                                             needs_layout_passes=False),
        scratch_types=[
            pltpu.VMEM((CPW, CH), jnp.int32),
            pltpu.VMEM((CPW, CH), jnp.int32),
            pltpu.VMEM((2, CH, HID), jnp.bfloat16),
            pltpu.VMEM((2, CH, HID), jnp.bfloat16),
            pltpu.VMEM((2, CH, dw), jnp.float32),
            pltpu.VMEM_SHARED((NR, dw), jnp.float32),
            pltpu.VMEM_SHARED((NR, HID), jnp.bfloat16),
            pltpu.SemaphoreType.DMA,
            pltpu.SemaphoreType.DMA,
            pltpu.SemaphoreType.DMA,
            pltpu.SemaphoreType.DMA,
            pltpu.SemaphoreType.DMA,
            pltpu.SemaphoreType.DMA,
        ],
    )
    def sc_layer(y_hbm, tp_hbm, src_hbm, dst_hbm, out_hbm,
                 srcv, dstv, tpv, gv, msgv, aggsh, ysh,
                 sem_t0, sem_t1, sem_g0, sem_g1, sem_s0, sem_s1):
        ci = lax.axis_index("c")
        si = lax.axis_index("s")
        w = ci * 16 + si
        nk = dw // 16

        @pl.loop(0, CH)
        def _(i):
            for k in range(nk):
                msgv[0, i, pl.ds(16 * k, 16)] = jnp.zeros((16,), jnp.float32)

        @pl.loop(0, RPT // CH)
        def _(i):
            base = (si * (RPT // CH) + i) * CH
            pltpu.sync_copy(msgv.at[0], aggsh.at[pl.ds(base, CH)])

        pltpu.sync_copy(src_hbm.at[w], srcv)
        pltpu.sync_copy(dst_hbm.at[w], dstv)

        @pl.loop(0, RPT // CH)
        def _(i):
            rows = pl.ds((si * (RPT // CH) + i) * CH, CH)
            pltpu.sync_copy(y_hbm.at[rows], gv.at[0])
            pltpu.sync_copy(gv.at[0], ysh.at[rows])

        if dw > HID:
            one0 = jnp.where(lax.iota(jnp.int32, 16) == 0,
                             1.0, 0.0).astype(jnp.float32)
            zer = jnp.zeros((16,), jnp.float32)

            @pl.loop(0, CH)
            def _(i):
                for b in range(2):
                    msgv[b, i, pl.ds(HID, 16)] = one0
                    for k in range(HID // 16 + 1, nk):
                        msgv[b, i, pl.ds(16 * k, 16)] = zer

        plsc.subcore_barrier()

        sems_t = (sem_t0, sem_t1)
        sems_g = (sem_g0, sem_g1)
        sems_s = (sem_s0, sem_s1)

        def start(j, b):
            pltpu.async_copy(tp_hbm.at[pl.ds((w * CPW + j) * CH, CH)],
                             tpv.at[b], sems_t[b])
            pltpu.async_copy(ysh.at[srcv.at[j]], gv.at[b], sems_g[b])

        def drain_scatter(b):
            pltpu.make_async_copy(out_hbm.at[0, pl.ds(0, CH)],
                                  msgv.at[b], sems_s[b]).wait()

        start(0, 0)

        @pl.loop(0, CPW, step=2)
        def _(g):
            for b in range(2):
                j = g + b
                nj = j + 1

                @pl.when(nj < CPW)
                def _():
                    start(nj, 1 - b)

                pltpu.make_async_copy(tp_hbm.at[pl.ds(0, CH)],
                                      tpv.at[b], sems_t[b]).wait()
                pltpu.make_async_copy(y_hbm.at[pl.ds(0, CH)],
                                      gv.at[b], sems_g[b]).wait()

                @plsc.parallel_loop(0, CH, unroll=4)
                def _(i):
                    for k in range(HID // 32):
                        s32 = pl.ds(32 * k, 32)
                        t_lo, t_hi = plsc.unpack(
                            tpv[b, i, s32], format=plsc.PackFormat.INTERLEAVED)
                        g_lo, g_hi = plsc.unpack(
                            gv[b, i, s32], format=plsc.PackFormat.INTERLEAVED)
                        msgv[b, i, pl.ds(32 * k, 16)] = jnp.maximum(
                            g_lo + t_lo, 0.0)
                        msgv[b, i, pl.ds(32 * k + 16, 16)] = jnp.maximum(
                            g_hi + t_hi, 0.0)

                @pl.when(j >= 1)
                def _():
                    drain_scatter(1 - b)

                pltpu.async_copy(msgv.at[b], aggsh.at[dstv.at[j]],
                                 sems_s[b], add=True)

        drain_scatter(1)
        plsc.subcore_barrier()

        @pl.loop(0, RPT // CH)
        def _(i):
            base = (si * (RPT // CH) + i) * CH
            pltpu.sync_copy(aggsh.at[pl.ds(base, CH)], msgv.at[0])
            pltpu.sync_copy(msgv.at[0], out_hbm.at[ci, pl.ds(base, CH)])

    return sc_layer


_sc_layer0 = _make_sc_layer(80)
_sc_layer1 = _make_sc_layer(64)


_BF16_PERM32 = np.empty(32, np.int32)
_BF16_PERM32[0::2] = np.arange(16)
_BF16_PERM32[1::2] = np.arange(16) + 16
_BF16_PERM64 = np.concatenate([_BF16_PERM32, _BF16_PERM32 + 32])


def kernel(x, edge_index, timestamps, time_w, time_b, W0, b0, W1, b1, Wc, bc):
    pad = EP - E
    src = edge_index[0].astype(jnp.int32)
    dst = edge_index[1].astype(jnp.int32)
    src_p = jnp.concatenate(
        [src, jnp.zeros((pad,), jnp.int32)]).reshape(NW, CPW, CH)
    dst_p = jnp.concatenate(
        [dst, N + (jnp.arange(pad, dtype=jnp.int32) % (NR - N))]
    ).reshape(NW, CPW, CH)
    ts_p = jnp.concatenate(
        [timestamps.astype(jnp.float32), jnp.zeros((pad,), jnp.float32)])
    t_rep = jnp.broadcast_to(ts_p[:, None], (EP, TIME_DIM)).reshape(EP // 8, 128)

    w0x, w0t = W0[:IN_CH], W0[IN_CH:]
    w1x, w1t = W1[:HID], W1[HID:]
    p64 = _BF16_PERM64
    w0xp = w0x[:, p64]
    w1xp = w1x[:, p64]
    w0tp = w0t[:, p64]
    w1tp = w1t[:, p64]
    b0p = b0[p64]
    b1p = b1[p64]
    eye8 = jnp.eye(8, dtype=jnp.float32)
    w0_blk = jnp.kron(eye8, w0tp)
    w1_blk = jnp.kron(eye8, w1tp)
    w_tile = jnp.tile(time_w.astype(jnp.float32), 8).reshape(1, 128)
    b_tile = jnp.tile(time_b.astype(jnp.float32), 8).reshape(1, 128)
    b0_blk = jnp.tile(b0p.astype(jnp.float32), 8).reshape(1, 512)
    b1_blk = jnp.tile(b1p.astype(jnp.float32), 8).reshape(1, 512)

    tp0m, tp1m = _tp_call(t_rep, w_tile, b_tile, w0_blk, b0_blk, w1_blk, b1_blk)
    tp0 = tp0m.reshape(EP, HID)
    tp1 = tp1m.reshape(EP, HID)
    x_pad = jnp.concatenate(
        [x, jnp.zeros((NR - N, IN_CH), jnp.float32)])
    y0 = _mm_call(x_pad, w0xp)

    agg0 = _sc_layer0(y0, tp0, src_p, dst_p)
    y1, deg = _mid_call(agg0, w1xp)
    agg1 = _sc_layer1(y1, tp1, src_p, dst_p)
    out = _fin_call(agg1, deg, Wc, bc.reshape(1, OUT_CH))
    return out[:N]
